# Initial kernel scaffold; baseline (speedup 1.0000x reference)
import jax
import jax.numpy as jnp
from jax.experimental import pallas as pl


def kernel(xyz, feature):
    raise NotImplementedError("write your pallas kernel here")


# speedup vs baseline: 29.3328x; 29.3328x over previous
import functools

import jax
import jax.numpy as jnp
from jax import lax
from jax.experimental import pallas as pl
from jax.experimental.pallas import tpu as pltpu
from jax.experimental.pallas import tpu_sc as plsc

RATIO = 4
NUM_SC_CORES = 2
NUM_SUBCORES = 16
NUM_WORKERS = NUM_SC_CORES * NUM_SUBCORES
IDX_CHUNK = 128


def _fps_body(x_ref, y_ref, z_ref, idx_ref, sx_ref, sy_ref, sz_ref, dist_ref):
    B, N = x_ref.shape
    M = idx_ref.shape[1]
    CH = 128
    lane = lax.broadcasted_iota(jnp.int32, (B, N), 1)
    lane_ch = lax.broadcasted_iota(jnp.int32, (B, CH), 1)
    dist_ref[...] = jnp.full((B, N), 1e10, jnp.float32)

    def chunk_body(c, f):
        def step(j, st):
            f, ai, ax, ay, az = st
            X = x_ref[...]
            Y = y_ref[...]
            Z = z_ref[...]
            m = lane == f
            cx = jnp.sum(jnp.where(m, X, 0.0), axis=1, keepdims=True)
            cy = jnp.sum(jnp.where(m, Y, 0.0), axis=1, keepdims=True)
            cz = jnp.sum(jnp.where(m, Z, 0.0), axis=1, keepdims=True)
            sel = lane_ch == j
            ai = jnp.where(sel, f, ai)
            ax = jnp.where(sel, cx, ax)
            ay = jnp.where(sel, cy, ay)
            az = jnp.where(sel, cz, az)
            d = (X - cx) ** 2 + (Y - cy) ** 2 + (Z - cz) ** 2
            dist = jnp.minimum(dist_ref[...], d)
            dist_ref[...] = dist
            rowmax = jnp.max(dist, axis=1, keepdims=True)
            f_next = jnp.min(jnp.where(dist == rowmax, lane, N), axis=1,
                             keepdims=True)
            return (f_next, ai, ax, ay, az)

        zi = jnp.zeros((B, CH), jnp.int32)
        zf = jnp.zeros((B, CH), jnp.float32)
        f, ai, ax, ay, az = lax.fori_loop(0, CH, step, (f, zi, zf, zf, zf))
        base = pl.multiple_of(c * CH, CH)
        idx_ref[:, pl.ds(base, CH)] = ai
        sx_ref[:, pl.ds(base, CH)] = ax
        sy_ref[:, pl.ds(base, CH)] = ay
        sz_ref[:, pl.ds(base, CH)] = az
        return f

    lax.fori_loop(0, M // CH, chunk_body, jnp.zeros((B, 1), jnp.int32))


def _fps(x, y, z, M):
    B, N = x.shape
    out_shape = [
        jax.ShapeDtypeStruct((B, M), jnp.int32),
        jax.ShapeDtypeStruct((B, M), jnp.float32),
        jax.ShapeDtypeStruct((B, M), jnp.float32),
        jax.ShapeDtypeStruct((B, M), jnp.float32),
    ]
    return pl.pallas_call(
        _fps_body,
        out_shape=out_shape,
        scratch_shapes=[pltpu.VMEM((B, N), jnp.float32)],
    )(x, y, z)


def _sc_gather(feat2, idx, pairs, M, N):
    B = idx.shape[0]
    workers_per_batch = NUM_WORKERS // B
    ppw = pairs // NUM_WORKERS

    @functools.partial(
        pl.kernel,
        mesh=plsc.VectorSubcoreMesh(core_axis_name="c", subcore_axis_name="s"),
        out_type=jax.ShapeDtypeStruct((pairs, M), jnp.float32),
        scratch_types=[
            pltpu.VMEM((M,), jnp.int32),
            pltpu.VMEM((N,), jnp.float32),
            pltpu.VMEM((M,), jnp.float32),
            pltpu.SemaphoreType.DMA,
        ],
        compiler_params=pltpu.CompilerParams(needs_layout_passes=False),
    )
    def k(feat_hbm, idx_hbm, out_hbm, idx_v, row_v, out_v, sem):
        cid = lax.axis_index("c")
        sid = lax.axis_index("s")
        wid = sid * NUM_SC_CORES + cid
        b = wid // workers_per_batch
        pltpu.sync_copy(idx_hbm.at[b], idx_v)

        def pair_body(p, _):
            pair = wid * ppw + p
            pltpu.sync_copy(feat_hbm.at[pair], row_v)

            def gather_grp(j, _):
                sl = pl.ds(j * 16, 16)
                out_v[sl] = plsc.load_gather(row_v, [idx_v[sl]])
                return 0

            lax.fori_loop(0, M // 16, gather_grp, 0)
            pltpu.sync_copy(out_v, out_hbm.at[pair])
            return 0

        lax.fori_loop(0, ppw, pair_body, 0)

    return k(feat2, idx)


def kernel(xyz, feature):
    B, _, N = xyz.shape
    _, C, _ = feature.shape
    M = N // RATIO

    x = xyz[:, 0, :]
    y = xyz[:, 1, :]
    z = xyz[:, 2, :]
    idx, sx, sy, sz = _fps(x, y, z, M)
    sampled = jnp.stack([sx, sy, sz], axis=1)

    feat2 = feature.reshape(B * C, N)
    gathered = _sc_gather(feat2, idx, B * C, M, N)
    sampled_feature = gathered.reshape(B, C, M)
    return sampled, sampled_feature


# speedup vs baseline: 29.9539x; 1.0212x over previous
import functools

import jax
import jax.numpy as jnp
from jax import lax
from jax.experimental import pallas as pl
from jax.experimental.pallas import tpu as pltpu
from jax.experimental.pallas import tpu_sc as plsc

RATIO = 4
NUM_SC_CORES = 2
NUM_SUBCORES = 16
NUM_WORKERS = NUM_SC_CORES * NUM_SUBCORES
IDX_CHUNK = 128


def _fps_body(xyzr_ref, idx_ref, sx_ref, sy_ref, sz_ref, dist_ref):
    G, _, R, L = xyzr_ref.shape
    N = 2 * L
    M = idx_ref.shape[1]
    CH = 128
    sub = lax.broadcasted_iota(jnp.int32, (R, L), 0)
    lane = lax.broadcasted_iota(jnp.int32, (R, L), 1)
    flatf = ((sub % 2) * L + lane).astype(jnp.float32)
    lane_ch = lax.broadcasted_iota(jnp.int32, (R, CH), 1)
    even_row = (lax.broadcasted_iota(jnp.int32, (R, 1), 0) % 2) == 0
    dist_ref[...] = jnp.full((G, R, L), 1e10, jnp.float32)

    def pair_combine(t, op):
        dn = jnp.roll(t, 1, axis=0)
        up = jnp.roll(t, -1, axis=0)
        return jnp.where(even_row, op(t, up), op(t, dn))

    def phase_b(g, mx):
        dist = dist_ref[g]
        cand = jnp.where(dist == mx, flatf, float(N))
        return pair_combine(jnp.min(cand, axis=1, keepdims=True), jnp.minimum)

    def phase_a(g, fg, j, acc1, acc2):
        Xg = xyzr_ref[g, 0]
        Yg = xyzr_ref[g, 1]
        Zg = xyzr_ref[g, 2]
        m = flatf == fg
        cx = pair_combine(
            jnp.sum(jnp.where(m, Xg, 0.0), axis=1, keepdims=True), jnp.add)
        cy = pair_combine(
            jnp.sum(jnp.where(m, Yg, 0.0), axis=1, keepdims=True), jnp.add)
        cz = pair_combine(
            jnp.sum(jnp.where(m, Zg, 0.0), axis=1, keepdims=True), jnp.add)
        d = (Xg - cx) ** 2 + (Yg - cy) ** 2 + (Zg - cz) ** 2
        dist = jnp.minimum(dist_ref[g], d)
        dist_ref[g] = dist
        mx = pair_combine(jnp.max(dist, axis=1, keepdims=True), jnp.maximum)
        sel = lane_ch == j
        acc1 = jnp.where(sel, jnp.where(even_row, fg, cx), acc1)
        acc2 = jnp.where(sel, jnp.where(even_row, cy, cz), acc2)
        return mx, acc1, acc2

    def chunk_body(c, carry):
        def step(j, st):
            (mx0, f1), (a10, a20, a11, a21) = st
            f0 = phase_b(0, mx0)
            mx1, a11, a21 = phase_a(1, f1, j, a11, a21)
            nmx0, a10, a20 = phase_a(0, f0, j, a10, a20)
            nf1 = phase_b(1, mx1)
            return ((nmx0, nf1), (a10, a20, a11, a21))

        zf = jnp.zeros((R, CH), jnp.float32)
        carry, accs = lax.fori_loop(0, CH, step,
                                    (carry, (zf, zf, zf, zf)))
        base = pl.multiple_of(c * CH, CH)
        for g in range(G):
            a1, a2 = accs[2 * g], accs[2 * g + 1]
            for b in range(R // 2):
                row = (R // 2) * g + b
                idx_ref[row, pl.ds(base, CH)] = a1[2 * b].astype(jnp.int32)
                sx_ref[row, pl.ds(base, CH)] = a1[2 * b + 1]
                sy_ref[row, pl.ds(base, CH)] = a2[2 * b]
                sz_ref[row, pl.ds(base, CH)] = a2[2 * b + 1]
        return carry

    carry0 = (jnp.full((R, 1), 1e10, jnp.float32),
              jnp.zeros((R, 1), jnp.float32))
    lax.fori_loop(0, M // CH, chunk_body, carry0)


def _fps(xyzr, M):
    G, _, R, L = xyzr.shape
    B = G * (R // 2)
    out_shape = [
        jax.ShapeDtypeStruct((B, M), jnp.int32),
        jax.ShapeDtypeStruct((B, M), jnp.float32),
        jax.ShapeDtypeStruct((B, M), jnp.float32),
        jax.ShapeDtypeStruct((B, M), jnp.float32),
    ]
    return pl.pallas_call(
        _fps_body,
        out_shape=out_shape,
        scratch_shapes=[pltpu.VMEM((G, R, L), jnp.float32)],
    )(xyzr)


def _sc_gather(feat2, idx, pairs, M, N):
    B = idx.shape[0]
    workers_per_batch = NUM_WORKERS // B
    ppw = pairs // NUM_WORKERS

    @functools.partial(
        pl.kernel,
        mesh=plsc.VectorSubcoreMesh(core_axis_name="c", subcore_axis_name="s"),
        out_type=jax.ShapeDtypeStruct((pairs, M), jnp.float32),
        scratch_types=[
            pltpu.VMEM((M,), jnp.int32),
            pltpu.VMEM((N,), jnp.float32),
            pltpu.VMEM((M,), jnp.float32),
            pltpu.SemaphoreType.DMA,
        ],
        compiler_params=pltpu.CompilerParams(needs_layout_passes=False),
    )
    def k(feat_hbm, idx_hbm, out_hbm, idx_v, row_v, out_v, sem):
        cid = lax.axis_index("c")
        sid = lax.axis_index("s")
        wid = sid * NUM_SC_CORES + cid
        b = wid // workers_per_batch
        pltpu.sync_copy(idx_hbm.at[b], idx_v)

        def pair_body(p, _):
            pair = wid * ppw + p
            pltpu.sync_copy(feat_hbm.at[pair], row_v)

            def gather_grp(j, _):
                sl = pl.ds(j * 16, 16)
                out_v[sl] = plsc.load_gather(row_v, [idx_v[sl]])
                return 0

            lax.fori_loop(0, M // 16, gather_grp, 0)
            pltpu.sync_copy(out_v, out_hbm.at[pair])
            return 0

        lax.fori_loop(0, ppw, pair_body, 0)

    return k(feat2, idx)


def kernel(xyz, feature):
    B, _, N = xyz.shape
    _, C, _ = feature.shape
    M = N // RATIO

    xyzr = (xyz.reshape(2, B // 2, 3, 2, N // 2)
            .transpose(0, 2, 1, 3, 4)
            .reshape(2, 3, B, N // 2))
    idx, sx, sy, sz = _fps(xyzr, M)
    sampled = jnp.stack([sx, sy, sz], axis=1)

    feat2 = feature.reshape(B * C, N)
    gathered = _sc_gather(feat2, idx, B * C, M, N)
    sampled_feature = gathered.reshape(B, C, M)
    return sampled, sampled_feature


# speedup vs baseline: 30.1388x; 1.0062x over previous
import functools

import jax
import jax.numpy as jnp
from jax import lax
from jax.experimental import pallas as pl
from jax.experimental.pallas import tpu as pltpu
from jax.experimental.pallas import tpu_sc as plsc

RATIO = 4
NUM_SC_CORES = 2
NUM_SUBCORES = 16
NUM_WORKERS = NUM_SC_CORES * NUM_SUBCORES
IDX_CHUNK = 128


def _fps_body(xyzr_ref, idx_ref, sx_ref, sy_ref, sz_ref, dist_ref):
    G, _, R, L = xyzr_ref.shape
    N = 2 * L
    M = idx_ref.shape[1]
    CH = 128
    sub = lax.broadcasted_iota(jnp.int32, (R, L), 0)
    lane = lax.broadcasted_iota(jnp.int32, (R, L), 1)
    flatf = ((sub % 2) * L + lane).astype(jnp.float32)
    lane_ch = lax.broadcasted_iota(jnp.int32, (R, CH), 1)
    even_row = (lax.broadcasted_iota(jnp.int32, (R, 1), 0) % 2) == 0
    dist_ref[...] = jnp.full((G, R, L), 1e10, jnp.float32)

    def pair_combine(t, op):
        dn = jnp.roll(t, 1, axis=0)
        up = jnp.roll(t, -1, axis=0)
        return jnp.where(even_row, op(t, up), op(t, dn))

    def phase_b(g, mx):
        dist = dist_ref[g]
        cand = jnp.where(dist == mx, flatf, float(N))
        return pair_combine(jnp.min(cand, axis=1, keepdims=True), jnp.minimum)

    def phase_a(g, fg, j, acc1, acc2):
        Xg = xyzr_ref[g, 0]
        Yg = xyzr_ref[g, 1]
        Zg = xyzr_ref[g, 2]
        m = flatf == fg
        cx = pair_combine(
            jnp.sum(jnp.where(m, Xg, 0.0), axis=1, keepdims=True), jnp.add)
        cy = pair_combine(
            jnp.sum(jnp.where(m, Yg, 0.0), axis=1, keepdims=True), jnp.add)
        cz = pair_combine(
            jnp.sum(jnp.where(m, Zg, 0.0), axis=1, keepdims=True), jnp.add)
        d = (Xg - cx) ** 2 + (Yg - cy) ** 2 + (Zg - cz) ** 2
        dist = jnp.minimum(dist_ref[g], d)
        dist_ref[g] = dist
        mx = pair_combine(jnp.max(dist, axis=1, keepdims=True), jnp.maximum)
        sel = lane_ch == j
        acc1 = jnp.where(sel, jnp.where(even_row, fg, cx), acc1)
        acc2 = jnp.where(sel, jnp.where(even_row, cy, cz), acc2)
        return mx, acc1, acc2

    def chunk_body(c, carry):
        def step(j, st):
            (mx0, f1), (a10, a20, a11, a21) = st
            f0 = phase_b(0, mx0)
            mx1, a11, a21 = phase_a(1, f1, j, a11, a21)
            nmx0, a10, a20 = phase_a(0, f0, j, a10, a20)
            nf1 = phase_b(1, mx1)
            return ((nmx0, nf1), (a10, a20, a11, a21))

        zf = jnp.zeros((R, CH), jnp.float32)
        carry, accs = lax.fori_loop(0, CH, step,
                                    (carry, (zf, zf, zf, zf)))
        base = pl.multiple_of(c * CH, CH)
        for g in range(G):
            a1, a2 = accs[2 * g], accs[2 * g + 1]
            for b in range(R // 2):
                row = (R // 2) * g + b
                idx_ref[row, pl.ds(base, CH)] = a1[2 * b].astype(jnp.int32)
                sx_ref[row, pl.ds(base, CH)] = a1[2 * b + 1]
                sy_ref[row, pl.ds(base, CH)] = a2[2 * b]
                sz_ref[row, pl.ds(base, CH)] = a2[2 * b + 1]
        return carry

    carry0 = (jnp.full((R, 1), 1e10, jnp.float32),
              jnp.zeros((R, 1), jnp.float32))
    lax.fori_loop(0, M // CH, chunk_body, carry0)


def _fps(xyzr, M):
    G, _, R, L = xyzr.shape
    B = G * (R // 2)
    out_shape = [
        jax.ShapeDtypeStruct((B, M), jnp.int32),
        jax.ShapeDtypeStruct((B, M), jnp.float32),
        jax.ShapeDtypeStruct((B, M), jnp.float32),
        jax.ShapeDtypeStruct((B, M), jnp.float32),
    ]
    return pl.pallas_call(
        _fps_body,
        out_shape=out_shape,
        scratch_shapes=[pltpu.VMEM((G, R, L), jnp.float32)],
    )(xyzr)


def _sc_gather(feat2, idx, pairs, M, N):
    B = idx.shape[0]
    workers_per_batch = NUM_WORKERS // B
    ppw = pairs // NUM_WORKERS

    @functools.partial(
        pl.kernel,
        mesh=plsc.VectorSubcoreMesh(core_axis_name="c", subcore_axis_name="s"),
        out_type=jax.ShapeDtypeStruct((pairs, M), jnp.float32),
        scratch_types=[
            pltpu.VMEM((M,), jnp.int32),
            pltpu.VMEM((2, N), jnp.float32),
            pltpu.VMEM((2, M), jnp.float32),
            pltpu.SemaphoreType.DMA,
            pltpu.SemaphoreType.DMA,
            pltpu.SemaphoreType.DMA,
            pltpu.SemaphoreType.DMA,
        ],
        compiler_params=pltpu.CompilerParams(needs_layout_passes=False),
    )
    def k(feat_hbm, idx_hbm, out_hbm, idx_v, row_v, out_v,
          in_sem0, in_sem1, out_sem0, out_sem1):
        cid = lax.axis_index("c")
        sid = lax.axis_index("s")
        wid = sid * NUM_SC_CORES + cid
        base = wid * ppw
        b = wid // workers_per_batch
        pltpu.sync_copy(idx_hbm.at[b], idx_v)

        in_sems = (in_sem0, in_sem1)
        out_sems = (out_sem0, out_sem1)

        def in_cp(p, buf):
            return pltpu.make_async_copy(feat_hbm.at[base + p],
                                         row_v.at[buf], in_sems[buf])

        def out_cp(p, buf):
            return pltpu.make_async_copy(out_v.at[buf],
                                         out_hbm.at[base + p], out_sems[buf])

        def gather(buf):
            bufs = jnp.full((16,), buf, jnp.int32)

            def gather_grp(j, _):
                for kq in range(4):
                    sl = pl.ds(j * 64 + kq * 16, 16)
                    out_v[buf, sl] = plsc.load_gather(row_v,
                                                      [bufs, idx_v[sl]])
                return 0
            lax.fori_loop(0, M // 64, gather_grp, 0)

        in_cp(0, 0).start()
        in_cp(1, 1).start()
        nt = ppw // 2

        def pair_body(t, _):
            for buf in range(2):
                p = 2 * t + buf

                @pl.when(t > 0)
                def _():
                    out_cp(2 * (t - 1) + buf, buf).wait()

                in_cp(p, buf).wait()
                gather(buf)
                out_cp(p, buf).start()

                @pl.when(t + 1 < nt)
                def _():
                    in_cp(p + 2, buf).start()
            return 0

        lax.fori_loop(0, nt, pair_body, 0)
        out_cp(ppw - 2, 0).wait()
        out_cp(ppw - 1, 1).wait()

    return k(feat2, idx)


def kernel(xyz, feature):
    B, _, N = xyz.shape
    _, C, _ = feature.shape
    M = N // RATIO

    xyzr = (xyz.reshape(2, B // 2, 3, 2, N // 2)
            .transpose(0, 2, 1, 3, 4)
            .reshape(2, 3, B, N // 2))
    idx, sx, sy, sz = _fps(xyzr, M)
    sampled = jnp.stack([sx, sy, sz], axis=1)

    feat2 = feature.reshape(B * C, N)
    gathered = _sc_gather(feat2, idx, B * C, M, N)
    sampled_feature = gathered.reshape(B, C, M)
    return sampled, sampled_feature


# speedup vs baseline: 30.2849x; 1.0048x over previous
import functools

import jax
import jax.numpy as jnp
from jax import lax
from jax.experimental import pallas as pl
from jax.experimental.pallas import tpu as pltpu
from jax.experimental.pallas import tpu_sc as plsc

RATIO = 4
NUM_SC_CORES = 2
NUM_SUBCORES = 16
NUM_WORKERS = NUM_SC_CORES * NUM_SUBCORES


def _fps_body(xyzr_ref, idx_ref, sx_ref, sy_ref, sz_ref, dist_ref):
    G, _, R, L = xyzr_ref.shape
    N = 2 * L
    M = idx_ref.shape[1]
    CH = 128
    sub = lax.broadcasted_iota(jnp.int32, (R, L), 0)
    lane = lax.broadcasted_iota(jnp.int32, (R, L), 1)
    flatf = ((sub % 2) * L + lane).astype(jnp.float32)
    lane_ch = lax.broadcasted_iota(jnp.int32, (R, CH), 1)
    even_row = (lax.broadcasted_iota(jnp.int32, (R, 1), 0) % 2) == 0
    dist_ref[...] = jnp.full((G, R, L), 1e10, jnp.float32)

    def pair_combine(t, op):
        dn = jnp.roll(t, 1, axis=0)
        up = jnp.roll(t, -1, axis=0)
        return jnp.where(even_row, op(t, up), op(t, dn))

    def phase_b(g, mx):
        dist = dist_ref[g]
        cand = jnp.where(dist == mx, flatf, float(N))
        return pair_combine(jnp.min(cand, axis=1, keepdims=True), jnp.minimum)

    def phase_a(g, fg, j, acc1, acc2):
        Xg = xyzr_ref[g, 0]
        Yg = xyzr_ref[g, 1]
        Zg = xyzr_ref[g, 2]
        m = flatf == fg
        cx = pair_combine(
            jnp.sum(jnp.where(m, Xg, 0.0), axis=1, keepdims=True), jnp.add)
        cy = pair_combine(
            jnp.sum(jnp.where(m, Yg, 0.0), axis=1, keepdims=True), jnp.add)
        cz = pair_combine(
            jnp.sum(jnp.where(m, Zg, 0.0), axis=1, keepdims=True), jnp.add)
        d = (Xg - cx) ** 2 + (Yg - cy) ** 2 + (Zg - cz) ** 2
        dist = jnp.minimum(dist_ref[g], d)
        dist_ref[g] = dist
        mx = pair_combine(jnp.max(dist, axis=1, keepdims=True), jnp.maximum)
        sel = lane_ch == j
        acc1 = jnp.where(sel, jnp.where(even_row, fg, cx), acc1)
        acc2 = jnp.where(sel, jnp.where(even_row, cy, cz), acc2)
        return mx, acc1, acc2

    def chunk_body(c, carry):
        def step(j, st):
            (mx0, f1), (a10, a20, a11, a21) = st
            f0 = phase_b(0, mx0)
            mx1, a11, a21 = phase_a(1, f1, j, a11, a21)
            nmx0, a10, a20 = phase_a(0, f0, j, a10, a20)
            nf1 = phase_b(1, mx1)
            return ((nmx0, nf1), (a10, a20, a11, a21))

        zf = jnp.zeros((R, CH), jnp.float32)
        carry, accs = lax.fori_loop(0, CH, step,
                                    (carry, (zf, zf, zf, zf)))
        base = pl.multiple_of(c * CH, CH)
        for g in range(G):
            a1, a2 = accs[2 * g], accs[2 * g + 1]
            for b in range(R // 2):
                row = (R // 2) * g + b
                idx_ref[row, pl.ds(base, CH)] = a1[2 * b].astype(jnp.int32)
                sx_ref[row, pl.ds(base, CH)] = a1[2 * b + 1]
                sy_ref[row, pl.ds(base, CH)] = a2[2 * b]
                sz_ref[row, pl.ds(base, CH)] = a2[2 * b + 1]
        return carry

    carry0 = (jnp.full((R, 1), 1e10, jnp.float32),
              jnp.zeros((R, 1), jnp.float32))
    lax.fori_loop(0, M // CH, chunk_body, carry0)


def _fps(xyzr, M):
    G, _, R, L = xyzr.shape
    B = G * (R // 2)
    out_shape = [
        jax.ShapeDtypeStruct((B, M), jnp.int32),
        jax.ShapeDtypeStruct((B, M), jnp.float32),
        jax.ShapeDtypeStruct((B, M), jnp.float32),
        jax.ShapeDtypeStruct((B, M), jnp.float32),
    ]
    return pl.pallas_call(
        _fps_body,
        out_shape=out_shape,
        scratch_shapes=[pltpu.VMEM((G, R, L), jnp.float32)],
    )(xyzr)


def _sc_gather(feat2, idx, pairs, M, N):
    B = idx.shape[0]
    workers_per_batch = NUM_WORKERS // B
    ppw = pairs // NUM_WORKERS

    @functools.partial(
        pl.kernel,
        mesh=plsc.VectorSubcoreMesh(core_axis_name="c", subcore_axis_name="s"),
        out_type=jax.ShapeDtypeStruct((pairs, M), jnp.float32),
        scratch_types=[
            pltpu.VMEM((M,), jnp.int32),
            pltpu.VMEM((2, N), jnp.float32),
            pltpu.VMEM((2, M), jnp.float32),
            pltpu.SemaphoreType.DMA,
            pltpu.SemaphoreType.DMA,
            pltpu.SemaphoreType.DMA,
            pltpu.SemaphoreType.DMA,
        ],
        compiler_params=pltpu.CompilerParams(needs_layout_passes=False),
    )
    def k(feat_hbm, idx_hbm, out_hbm, idx_v, row_v, out_v,
          in_sem0, in_sem1, out_sem0, out_sem1):
        cid = lax.axis_index("c")
        sid = lax.axis_index("s")
        wid = sid * NUM_SC_CORES + cid
        base = wid * ppw
        b = wid // workers_per_batch
        pltpu.sync_copy(idx_hbm.at[b], idx_v)

        in_sems = (in_sem0, in_sem1)
        out_sems = (out_sem0, out_sem1)

        def in_cp(p, buf):
            return pltpu.make_async_copy(feat_hbm.at[base + p],
                                         row_v.at[buf], in_sems[buf])

        def out_cp(p, buf):
            return pltpu.make_async_copy(out_v.at[buf],
                                         out_hbm.at[base + p], out_sems[buf])

        def gather(buf):
            bufs = jnp.full((16,), buf, jnp.int32)

            def gather_grp(j, _):
                for kq in range(4):
                    sl = pl.ds(j * 64 + kq * 16, 16)
                    out_v[buf, sl] = plsc.load_gather(row_v,
                                                      [bufs, idx_v[sl]])
                return 0
            lax.fori_loop(0, M // 64, gather_grp, 0)

        in_cp(0, 0).start()
        in_cp(1, 1).start()
        nt = ppw // 2

        def pair_body(t, _):
            for buf in range(2):
                p = 2 * t + buf

                @pl.when(t > 0)
                def _():
                    out_cp(2 * (t - 1) + buf, buf).wait()

                in_cp(p, buf).wait()
                gather(buf)
                out_cp(p, buf).start()

                @pl.when(t + 1 < nt)
                def _():
                    in_cp(p + 2, buf).start()
            return 0

        lax.fori_loop(0, nt, pair_body, 0)
        out_cp(ppw - 2, 0).wait()
        out_cp(ppw - 1, 1).wait()

    return k(feat2, idx)


def kernel(xyz, feature):
    B, _, N = xyz.shape
    _, C, _ = feature.shape
    M = N // RATIO

    xyzr = (xyz.reshape(2, B // 2, 3, 2, N // 2)
            .transpose(0, 2, 1, 3, 4)
            .reshape(2, 3, B, N // 2))
    idx, sx, sy, sz = _fps(xyzr, M)
    sampled = jnp.stack([sx, sy, sz], axis=1)

    feat2 = feature.reshape(B * C, N)
    gathered = _sc_gather(feat2, idx, B * C, M, N)
    sampled_feature = gathered.reshape(B, C, M)
    return sampled, sampled_feature
